# baseline (speedup 1.0000x reference)
import functools

import jax
import jax.numpy as jnp
from jax import lax
from jax.experimental import pallas as pl
from jax.experimental.pallas import tpu as pltpu
from jax.experimental.pallas import tpu_sc as plsc

_N = 96
_B, _HG, _WG = 4, 256, 256
_P = _B * _HG * _WG
_NW = 32
_PTS_PER_W = _P // _NW
_C = 32
_CHUNKS = _PTS_PER_W // _C
_TAPS = 16
_ROWS = _TAPS * _C
_BLK = 1024


def _sc_sample(table, ux, uy):
    mesh = plsc.VectorSubcoreMesh(core_axis_name="c", subcore_axis_name="s")

    @functools.partial(
        pl.kernel,
        out_type=jax.ShapeDtypeStruct((_P * _N,), jnp.float32),
        mesh=mesh,
        compiler_params=pltpu.CompilerParams(use_tc_tiling_on_sc=False,
                                             needs_layout_passes=False),
        scratch_types=[
            pltpu.VMEM((_BLK,), jnp.float32),
            pltpu.VMEM((_BLK,), jnp.float32),
            pltpu.VMEM((_ROWS,), jnp.int32),
            pltpu.VMEM((_ROWS,), jnp.int32),
            pltpu.VMEM((_ROWS,), jnp.float32),
            pltpu.VMEM((_ROWS,), jnp.float32),
            pltpu.VMEM((_ROWS, _N), jnp.bfloat16),
            pltpu.VMEM((_ROWS, _N), jnp.bfloat16),
            pltpu.VMEM((_C * _N,), jnp.float32),
            pltpu.VMEM((_C * _N,), jnp.float32),
            pltpu.SemaphoreType.DMA,
            pltpu.SemaphoreType.DMA,
            pltpu.SemaphoreType.DMA,
            pltpu.SemaphoreType.DMA,
        ],
    )
    def tex_kernel(table_hbm, ux_hbm, uy_hbm, out_hbm,
                   ux_v, uy_v, idx_a, idx_b, w_a, w_b, taps_a, taps_b,
                   out_a, out_b, sem_a, sem_b, osem_a, osem_b):
        wid = lax.axis_index("s") * 2 + lax.axis_index("c")
        pbase = wid * _PTS_PER_W

        def stage(i, blkpos, idx_v, w_v):
            iota = lax.iota(jnp.int32, 16)
            lane_l = jnp.right_shift(iota, 2)
            wpitch_v = jnp.right_shift(iota * 0 + 512, lane_l)
            wm2_v = wpitch_v - 2
            sx_v = (wpitch_v - 1).astype(jnp.float32) * 0.5
            off_v = 349525 - jnp.right_shift(iota * 0 + 349525, 2 * lane_l)
            dx_v = jnp.bitwise_and(iota, 1)
            dy_v = jnp.bitwise_and(jnp.right_shift(iota, 1), 1)
            maskx = dx_v == 1
            masky = dy_v == 1

            for g in range(_C // 16):
                xs = ux_v[pl.ds(blkpos + g * 16, 16)]
                ys = uy_v[pl.ds(blkpos + g * 16, 16)]
                for pp in range(16):
                    p = g * 16 + pp
                    fx = (xs[pp] + 1.0) * sx_v
                    fy = (ys[pp] + 1.0) * sx_v
                    x0 = jnp.minimum(fx.astype(jnp.int32), wm2_v)
                    y0 = jnp.minimum(fy.astype(jnp.int32), wm2_v)
                    wx1 = fx - x0.astype(jnp.float32)
                    wy1 = fy - y0.astype(jnp.float32)
                    wxs = jnp.where(maskx, wx1, 1.0 - wx1)
                    wys = jnp.where(masky, wy1, 1.0 - wy1)
                    idx_v[pl.ds(p * _TAPS, 16)] = (
                        (y0 + dy_v) * wpitch_v + (x0 + dx_v) + off_v)
                    w_v[pl.ds(p * _TAPS, 16)] = wxs * wys

        def copies(idx_v, taps_v, sem):
            return [
                pltpu.make_async_copy(table_hbm.at[idx_v.at[pl.ds(j * 128, 128)]],
                                      taps_v.at[pl.ds(j * 128, 128)],
                                      sem)
                for j in range(_ROWS // 128)
            ]

        def issue(idx_v, taps_v, sem):
            for cp in copies(idx_v, taps_v, sem):
                cp.start()

        def drain(idx_v, taps_v, sem):
            for cp in copies(idx_v, taps_v, sem):
                cp.wait()

        def out_copy(i, out_v, osem):
            return pltpu.make_async_copy(
                out_v, out_hbm.at[pl.ds((pbase + i * _C) * _N, _C * _N)], osem)

        def accumulate(i, w_v, taps_v, out_v, osem):
            def point_body(q, cc):
                for pu in range(4):
                    p = q * 4 + pu
                    wv = w_v[pl.ds(p * _TAPS, 16)]
                    accs = [None] * (_N // 16)
                    for t in range(_TAPS):
                        w = wv[t]
                        for k2 in range(_N // 32):
                            tap32 = taps_v[p * _TAPS + t, pl.ds(k2 * 32, 32)]
                            lo, hi = plsc.unpack(
                                tap32, format=plsc.PackFormat.INTERLEAVED,
                                preferred_element_type=jnp.float32)
                            ka, kb = 2 * k2, 2 * k2 + 1
                            if t == 0:
                                accs[ka] = w * lo
                                accs[kb] = w * hi
                            else:
                                accs[ka] = accs[ka] + w * lo
                                accs[kb] = accs[kb] + w * hi
                    for k in range(_N // 16):
                        out_v[pl.ds(p * _N + k * 16, 16)] = accs[k]
                return cc

            lax.fori_loop(0, _C // 4, point_body, 0)
            out_copy(i, out_v, osem).start()

        pltpu.sync_copy(ux_hbm.at[pl.ds(pbase, _BLK)], ux_v)
        pltpu.sync_copy(uy_hbm.at[pl.ds(pbase, _BLK)], uy_v)
        stage(0, 0, idx_a, w_a)
        issue(idx_a, taps_a, sem_a)

        def body(j, carry):
            ia = 2 * j
            ib = 2 * j + 1
            jm = jnp.bitwise_and(j, 15)
            bpos_a = jm * (2 * _C)
            bpos_b = bpos_a + _C

            stage(ib, bpos_b, idx_b, w_b)
            issue(idx_b, taps_b, sem_b)
            drain(idx_a, taps_a, sem_a)

            @pl.when(j > 0)
            def _():
                out_copy(ia, out_a, osem_a).wait()

            accumulate(ia, w_a, taps_a, out_a, osem_a)

            @pl.when(jnp.bitwise_and(j + 1, 15) == 0)
            def _():
                pltpu.sync_copy(
                    ux_hbm.at[pl.ds(pbase + (j + 1) * (2 * _C), _BLK)], ux_v)
                pltpu.sync_copy(
                    uy_hbm.at[pl.ds(pbase + (j + 1) * (2 * _C), _BLK)], uy_v)

            @pl.when(ib + 1 < _CHUNKS)
            def _():
                stage(ib + 1, jnp.bitwise_and(j + 1, 15) * (2 * _C), idx_a, w_a)
                issue(idx_a, taps_a, sem_a)

            drain(idx_b, taps_b, sem_b)

            @pl.when(j > 0)
            def _():
                out_copy(ib, out_b, osem_b).wait()

            accumulate(ib, w_b, taps_b, out_b, osem_b)
            return carry

        lax.fori_loop(0, _CHUNKS // 2, body, 0)
        out_copy(_CHUNKS - 2, out_a, osem_a).wait()
        out_copy(_CHUNKS - 1, out_b, osem_b).wait()

    return tex_kernel(table, ux, uy)


def kernel(uv, layer1, layer2, layer3, layer4):
    perm = [32 * b + o for b in range(_N // 32)
            for pair in zip(range(16), range(16, 32)) for o in pair]
    tabs = [l[0].reshape(_N, -1)[perm, :].T.astype(jnp.bfloat16)
            for l in (layer1, layer2, layer3, layer4)]
    table = jnp.concatenate(tabs, axis=0)
    ux = uv[..., 0].reshape(-1)
    uy = uv[..., 1].reshape(-1)
    out = _sc_sample(table, ux, uy)
    return out.reshape(_B, _HG, _WG, _N).transpose(0, 3, 1, 2)


# speedup vs baseline: 1.2832x; 1.2832x over previous
import functools

import jax
import jax.numpy as jnp
from jax import lax
from jax.experimental import pallas as pl
from jax.experimental.pallas import tpu as pltpu
from jax.experimental.pallas import tpu_sc as plsc

_N = 96
_B, _HG, _WG = 4, 256, 256
_P = _B * _HG * _WG
_NW = 32
_PTS_PER_W = _P // _NW
_C = 32
_CHUNKS = _PTS_PER_W // _C
_TAPS = 16
_ROWS = _TAPS * _C
_BLK = 1024


def _sc_sample(table, ux, uy):
    mesh = plsc.VectorSubcoreMesh(core_axis_name="c", subcore_axis_name="s")

    @functools.partial(
        pl.kernel,
        out_type=jax.ShapeDtypeStruct((_P * _N,), jnp.float32),
        mesh=mesh,
        compiler_params=pltpu.CompilerParams(use_tc_tiling_on_sc=False),
        scratch_types=[
            pltpu.VMEM((_BLK,), jnp.float32),
            pltpu.VMEM((_BLK,), jnp.float32),
            pltpu.VMEM((_ROWS,), jnp.int32),
            pltpu.VMEM((_ROWS,), jnp.int32),
            pltpu.VMEM((_ROWS,), jnp.float32),
            pltpu.VMEM((_ROWS,), jnp.float32),
            pltpu.VMEM((_ROWS, _N), jnp.float32),
            pltpu.VMEM((_ROWS, _N), jnp.float32),
            pltpu.VMEM((_C * _N,), jnp.float32),
            pltpu.VMEM((_C * _N,), jnp.float32),
            pltpu.SemaphoreType.DMA,
            pltpu.SemaphoreType.DMA,
            pltpu.SemaphoreType.DMA,
            pltpu.SemaphoreType.DMA,
        ],
    )
    def tex_kernel(table_hbm, ux_hbm, uy_hbm, out_hbm,
                   ux_v, uy_v, idx_a, idx_b, w_a, w_b, taps_a, taps_b,
                   out_a, out_b, sem_a, sem_b, osem_a, osem_b):
        wid = lax.axis_index("s") * 2 + lax.axis_index("c")
        pbase = wid * _PTS_PER_W

        def stage(i, blkpos, idx_v, w_v):
            iota = lax.iota(jnp.int32, 16)
            lane_l = jnp.right_shift(iota, 2)
            wpitch_v = jnp.right_shift(iota * 0 + 512, lane_l)
            wm2_v = wpitch_v - 2
            sx_v = (wpitch_v - 1).astype(jnp.float32) * 0.5
            off_v = 349525 - jnp.right_shift(iota * 0 + 349525, 2 * lane_l)
            dx_v = jnp.bitwise_and(iota, 1)
            dy_v = jnp.bitwise_and(jnp.right_shift(iota, 1), 1)
            maskx = dx_v == 1
            masky = dy_v == 1

            for g in range(_C // 16):
                xs = ux_v[pl.ds(blkpos + g * 16, 16)]
                ys = uy_v[pl.ds(blkpos + g * 16, 16)]
                for pp in range(16):
                    p = g * 16 + pp
                    fx = (xs[pp] + 1.0) * sx_v
                    fy = (ys[pp] + 1.0) * sx_v
                    x0 = jnp.minimum(fx.astype(jnp.int32), wm2_v)
                    y0 = jnp.minimum(fy.astype(jnp.int32), wm2_v)
                    wx1 = fx - x0.astype(jnp.float32)
                    wy1 = fy - y0.astype(jnp.float32)
                    wxs = jnp.where(maskx, wx1, 1.0 - wx1)
                    wys = jnp.where(masky, wy1, 1.0 - wy1)
                    idx_v[pl.ds(p * _TAPS, 16)] = (
                        (y0 + dy_v) * wpitch_v + (x0 + dx_v) + off_v)
                    w_v[pl.ds(p * _TAPS, 16)] = wxs * wys

        def copies(idx_v, taps_v, sem):
            return [
                pltpu.make_async_copy(table_hbm.at[idx_v.at[pl.ds(j * 128, 128)]],
                                      taps_v.at[pl.ds(j * 128, 128)],
                                      sem)
                for j in range(_ROWS // 128)
            ]

        def issue(idx_v, taps_v, sem):
            for cp in copies(idx_v, taps_v, sem):
                cp.start()

        def drain(idx_v, taps_v, sem):
            for cp in copies(idx_v, taps_v, sem):
                cp.wait()

        def out_copy(i, out_v, osem):
            return pltpu.make_async_copy(
                out_v, out_hbm.at[pl.ds((pbase + i * _C) * _N, _C * _N)], osem)

        def accumulate(i, w_v, taps_v, out_v, osem):
            def point_body(q, cc):
                for pu in range(8):
                    p = q * 8 + pu
                    wv = w_v[pl.ds(p * _TAPS, 16)]
                    accs = [None] * (_N // 16)
                    for t in range(_TAPS):
                        w = wv[t]
                        for k in range(_N // 16):
                            term = w * taps_v[p * _TAPS + t, pl.ds(k * 16, 16)]
                            accs[k] = term if t == 0 else accs[k] + term
                    for k in range(_N // 16):
                        out_v[pl.ds(p * _N + k * 16, 16)] = accs[k]
                return cc

            lax.fori_loop(0, _C // 8, point_body, 0)
            out_copy(i, out_v, osem).start()

        pltpu.sync_copy(ux_hbm.at[pl.ds(pbase, _BLK)], ux_v)
        pltpu.sync_copy(uy_hbm.at[pl.ds(pbase, _BLK)], uy_v)
        stage(0, 0, idx_a, w_a)
        issue(idx_a, taps_a, sem_a)

        def body(j, carry):
            ia = 2 * j
            ib = 2 * j + 1
            jm = jnp.bitwise_and(j, 15)
            bpos_a = jm * (2 * _C)
            bpos_b = bpos_a + _C

            stage(ib, bpos_b, idx_b, w_b)
            issue(idx_b, taps_b, sem_b)
            drain(idx_a, taps_a, sem_a)

            @pl.when(j > 0)
            def _():
                out_copy(ia, out_a, osem_a).wait()

            accumulate(ia, w_a, taps_a, out_a, osem_a)

            @pl.when(jnp.bitwise_and(j + 1, 15) == 0)
            def _():
                pltpu.sync_copy(
                    ux_hbm.at[pl.ds(pbase + (j + 1) * (2 * _C), _BLK)], ux_v)
                pltpu.sync_copy(
                    uy_hbm.at[pl.ds(pbase + (j + 1) * (2 * _C), _BLK)], uy_v)

            @pl.when(ib + 1 < _CHUNKS)
            def _():
                stage(ib + 1, jnp.bitwise_and(j + 1, 15) * (2 * _C), idx_a, w_a)
                issue(idx_a, taps_a, sem_a)

            drain(idx_b, taps_b, sem_b)

            @pl.when(j > 0)
            def _():
                out_copy(ib, out_b, osem_b).wait()

            accumulate(ib, w_b, taps_b, out_b, osem_b)
            return carry

        lax.fori_loop(0, _CHUNKS // 2, body, 0)
        out_copy(_CHUNKS - 2, out_a, osem_a).wait()
        out_copy(_CHUNKS - 1, out_b, osem_b).wait()

    return tex_kernel(table, ux, uy)


def kernel(uv, layer1, layer2, layer3, layer4):
    tabs = [l[0].reshape(_N, -1).T for l in (layer1, layer2, layer3, layer4)]
    table = jnp.concatenate(tabs, axis=0)
    ux = uv[..., 0].reshape(-1)
    uy = uv[..., 1].reshape(-1)
    out = _sc_sample(table, ux, uy)
    return out.reshape(_B, _HG, _WG, _N).transpose(0, 3, 1, 2)


# speedup vs baseline: 1.4912x; 1.1621x over previous
import functools

import jax
import jax.numpy as jnp
from jax import lax
from jax.experimental import pallas as pl
from jax.experimental.pallas import tpu as pltpu
from jax.experimental.pallas import tpu_sc as plsc

_N = 96
_B, _HG, _WG = 4, 256, 256
_P = _B * _HG * _WG
_NW = 32
_PTS_PER_W = _P // _NW
_C = 32
_CHUNKS = _PTS_PER_W // _C
_TAPS = 16
_ROWS = _TAPS * _C
_BLK = 1024


def _sc_sample(table, ux, uy):
    mesh = plsc.VectorSubcoreMesh(core_axis_name="c", subcore_axis_name="s")

    @functools.partial(
        pl.kernel,
        out_type=jax.ShapeDtypeStruct((_P * _N,), jnp.float32),
        mesh=mesh,
        compiler_params=pltpu.CompilerParams(use_tc_tiling_on_sc=False),
        scratch_types=[
            pltpu.VMEM((_BLK,), jnp.float32),
            pltpu.VMEM((_BLK,), jnp.float32),
            pltpu.VMEM((_ROWS,), jnp.int32),
            pltpu.VMEM((_ROWS,), jnp.int32),
            pltpu.VMEM((_ROWS,), jnp.float32),
            pltpu.VMEM((_ROWS,), jnp.float32),
            pltpu.VMEM((_ROWS, _N), jnp.float32),
            pltpu.VMEM((_ROWS, _N), jnp.float32),
            pltpu.VMEM((_C * _N,), jnp.float32),
            pltpu.VMEM((_C * _N,), jnp.float32),
            pltpu.SemaphoreType.DMA,
            pltpu.SemaphoreType.DMA,
            pltpu.SemaphoreType.DMA,
            pltpu.SemaphoreType.DMA,
        ],
    )
    def tex_kernel(table_hbm, ux_hbm, uy_hbm, out_hbm,
                   ux_v, uy_v, idx_a, idx_b, w_a, w_b, taps_a, taps_b,
                   out_a, out_b, sem_a, sem_b, osem_a, osem_b):
        wid = lax.axis_index("s") * 2 + lax.axis_index("c")
        pbase = wid * _PTS_PER_W

        def stage(i, blkpos, idx_v, w_v):
            iota = lax.iota(jnp.int32, 16)
            lane_l = jnp.right_shift(iota, 2)
            wpitch_v = jnp.right_shift(iota * 0 + 512, lane_l)
            wm2_v = wpitch_v - 2
            sx_v = (wpitch_v - 1).astype(jnp.float32) * 0.5
            off_v = 349525 - jnp.right_shift(iota * 0 + 349525, 2 * lane_l)
            dx_v = jnp.bitwise_and(iota, 1)
            dy_v = jnp.bitwise_and(jnp.right_shift(iota, 1), 1)
            maskx = dx_v == 1
            masky = dy_v == 1

            for g in range(_C // 16):
                xs = ux_v[pl.ds(blkpos + g * 16, 16)]
                ys = uy_v[pl.ds(blkpos + g * 16, 16)]
                for pp in range(16):
                    p = g * 16 + pp
                    fx = (xs[pp] + 1.0) * sx_v
                    fy = (ys[pp] + 1.0) * sx_v
                    x0 = jnp.minimum(fx.astype(jnp.int32), wm2_v)
                    y0 = jnp.minimum(fy.astype(jnp.int32), wm2_v)
                    wx1 = fx - x0.astype(jnp.float32)
                    wy1 = fy - y0.astype(jnp.float32)
                    wxs = jnp.where(maskx, wx1, 1.0 - wx1)
                    wys = jnp.where(masky, wy1, 1.0 - wy1)
                    idx_v[pl.ds(p * _TAPS, 16)] = (
                        (y0 + dy_v) * wpitch_v + (x0 + dx_v) + off_v)
                    w_v[pl.ds(p * _TAPS, 16)] = wxs * wys

        def copies(idx_v, taps_v, sem):
            return [
                pltpu.make_async_copy(table_hbm.at[idx_v.at[pl.ds(j * 128, 128)]],
                                      taps_v.at[pl.ds(j * 128, 128)],
                                      sem)
                for j in range(_ROWS // 128)
            ]

        def issue(idx_v, taps_v, sem):
            for cp in copies(idx_v, taps_v, sem):
                cp.start()

        def drain(idx_v, taps_v, sem):
            for cp in copies(idx_v, taps_v, sem):
                cp.wait()

        def out_copy(i, out_v, osem):
            return pltpu.make_async_copy(
                out_v, out_hbm.at[pl.ds((pbase + i * _C) * _N, _C * _N)], osem)

        def accumulate(i, w_v, taps_v, out_v, osem):
            def point_body(q, cc):
                for pu in range(4):
                    p = q * 4 + pu
                    wv = w_v[pl.ds(p * _TAPS, 16)]
                    accs = [None] * (_N // 16)
                    for t in range(_TAPS):
                        w = wv[t]
                        for k in range(_N // 16):
                            term = w * taps_v[p * _TAPS + t, pl.ds(k * 16, 16)]
                            accs[k] = term if t == 0 else accs[k] + term
                    for k in range(_N // 16):
                        out_v[pl.ds(p * _N + k * 16, 16)] = accs[k]
                return cc

            lax.fori_loop(0, _C // 4, point_body, 0)
            out_copy(i, out_v, osem).start()

        pltpu.sync_copy(ux_hbm.at[pl.ds(pbase, _BLK)], ux_v)
        pltpu.sync_copy(uy_hbm.at[pl.ds(pbase, _BLK)], uy_v)
        stage(0, 0, idx_a, w_a)
        issue(idx_a, taps_a, sem_a)

        def body(j, carry):
            ia = 2 * j
            ib = 2 * j + 1
            jm = jnp.bitwise_and(j, 15)
            bpos_a = jm * (2 * _C)
            bpos_b = bpos_a + _C

            stage(ib, bpos_b, idx_b, w_b)
            issue(idx_b, taps_b, sem_b)
            drain(idx_a, taps_a, sem_a)

            @pl.when(j > 0)
            def _():
                out_copy(ia, out_a, osem_a).wait()

            accumulate(ia, w_a, taps_a, out_a, osem_a)

            @pl.when(jnp.bitwise_and(j + 1, 15) == 0)
            def _():
                pltpu.sync_copy(
                    ux_hbm.at[pl.ds(pbase + (j + 1) * (2 * _C), _BLK)], ux_v)
                pltpu.sync_copy(
                    uy_hbm.at[pl.ds(pbase + (j + 1) * (2 * _C), _BLK)], uy_v)

            @pl.when(ib + 1 < _CHUNKS)
            def _():
                stage(ib + 1, jnp.bitwise_and(j + 1, 15) * (2 * _C), idx_a, w_a)
                issue(idx_a, taps_a, sem_a)

            drain(idx_b, taps_b, sem_b)

            @pl.when(j > 0)
            def _():
                out_copy(ib, out_b, osem_b).wait()

            accumulate(ib, w_b, taps_b, out_b, osem_b)
            return carry

        lax.fori_loop(0, _CHUNKS // 2, body, 0)
        out_copy(_CHUNKS - 2, out_a, osem_a).wait()
        out_copy(_CHUNKS - 1, out_b, osem_b).wait()

    return tex_kernel(table, ux, uy)


def kernel(uv, layer1, layer2, layer3, layer4):
    tabs = [l[0].reshape(_N, -1).T for l in (layer1, layer2, layer3, layer4)]
    table = jnp.concatenate(tabs, axis=0)
    ux = uv[..., 0].reshape(-1)
    uy = uv[..., 1].reshape(-1)
    out = _sc_sample(table, ux, uy)
    return out.reshape(_B, _HG, _WG, _N).transpose(0, 3, 1, 2)
